# baseline (speedup 1.0000x reference)
import functools

import jax
import jax.numpy as jnp
from jax import lax
from jax.experimental import pallas as pl
from jax.experimental.pallas import tpu as pltpu, tpu_sc as plsc

N = 10000
E = 320000
C_IN = 128
H = 128
C_OUT = 128

BLK = 2000

NC, NS = 2, 16
NW = NC * NS
K = 128
G = E // K
GPW = G // NW
GEXTRA = G - GPW * NW
NPAD = 10112
RPT = NPAD // NS


def _pre_body(x_ref, w1_ref, b1_ref, w2_ref, b2_ref, wc_ref, bc_ref,
              h_ref, msg_ref):
    f32 = jnp.float32
    h1 = jnp.maximum(
        jnp.dot(x_ref[...], w1_ref[...], preferred_element_type=f32)
        + b1_ref[...], 0.0)
    h = jnp.dot(h1, w2_ref[...], preferred_element_type=f32) + b2_ref[...]
    h_ref[...] = h
    msg_ref[...] = jnp.maximum(
        jnp.dot(h, wc_ref[...], preferred_element_type=f32) + bc_ref[...], 0.0)


def _pre(x, W1, b1, W2, b2, Wc, bc):
    full = lambda shape: pl.BlockSpec(shape, lambda i: (0, 0))
    row = lambda shape: pl.BlockSpec(shape, lambda i: (i, 0))
    return pl.pallas_call(
        _pre_body,
        grid=(N // BLK,),
        in_specs=[row((BLK, C_IN)), full((C_IN, H)), full((1, H)),
                  full((H, H)), full((1, H)), full((H, H)), full((1, H))],
        out_specs=[row((BLK, H)), row((BLK, H))],
        out_shape=[jax.ShapeDtypeStruct((N, H), jnp.float32),
                   jax.ShapeDtypeStruct((N, H), jnp.float32)],
        compiler_params=pltpu.CompilerParams(
            dimension_semantics=("parallel",)),
    )(x, W1, b1, W2, b2, Wc, bc)


def _post_body(h_ref, a0_ref, a1_ref, w3a_ref, w3b_ref, b3_ref,
               w4_ref, b4_ref, o_ref):
    f32 = jnp.float32
    agg = a0_ref[0] + a1_ref[0]
    t = (jnp.dot(h_ref[...], w3a_ref[...], preferred_element_type=f32)
         + jnp.dot(agg, w3b_ref[...], preferred_element_type=f32)
         + b3_ref[...])
    t = jnp.maximum(t, 0.0)
    o_ref[...] = jnp.tanh(
        jnp.dot(t, w4_ref[...], preferred_element_type=f32) + b4_ref[...])


def _post(h, agg2, W3a, W3b, b3, W4, b4):
    full = lambda shape: pl.BlockSpec(shape, lambda i: (0, 0))
    row = lambda shape: pl.BlockSpec(shape, lambda i: (i, 0))
    return pl.pallas_call(
        _post_body,
        grid=(N // BLK,),
        in_specs=[row((BLK, H)),
                  pl.BlockSpec((1, BLK, H), lambda i: (0, i, 0)),
                  pl.BlockSpec((1, BLK, H), lambda i: (1, i, 0)),
                  full((H, H)), full((H, H)), full((1, H)),
                  full((H, C_OUT)), full((1, C_OUT))],
        out_specs=row((BLK, C_OUT)),
        out_shape=jax.ShapeDtypeStruct((N, C_OUT), jnp.float32),
        compiler_params=pltpu.CompilerParams(
            dimension_semantics=("parallel",)),
    )(h, agg2, agg2, W3a, W3b, b3, W4, b4)


def _sc_body(msg_hbm, edge_hbm, out_hbm,
             idx_b, rows_v, agg_sp,
             sem_i0, sem_i1, sem_g0, sem_g1):
    cid = lax.axis_index("c")
    sid = lax.axis_index("s")
    wid = cid * NS + sid
    sem_i = (sem_i0, sem_i1)
    sem_g = (sem_g0, sem_g1)

    zeros16 = jnp.zeros((16,), jnp.float32)

    def zrow(i, carry):
        def zcol(j, c2):
            rows_v[0, i, pl.ds(j * 16, 16)] = zeros16
            return c2
        return lax.fori_loop(0, H // 16, zcol, carry)

    lax.fori_loop(0, K, zrow, 0)

    def zcopy(i, carry):
        pltpu.sync_copy(rows_v.at[0, pl.ds(0, K)],
                        agg_sp.at[pl.ds(sid * RPT + i * K, K)])
        return carry

    lax.fori_loop(0, RPT // K, zcopy, 0)
    pltpu.sync_copy(rows_v.at[0, pl.ds(0, RPT - (RPT // K) * K)],
                    agg_sp.at[pl.ds(sid * RPT + (RPT // K) * K,
                                    RPT - (RPT // K) * K)])

    gbase = wid * GPW

    def idx_start(g, s):
        pltpu.async_copy(edge_hbm.at[:, pl.ds(g * K, K)], idx_b.at[s],
                         sem_i[s])

    def idx_wait(s):
        pltpu.make_async_copy(edge_hbm.at[:, pl.ds(0, K)], idx_b.at[s],
                              sem_i[s]).wait()

    def gather_start(s):
        pltpu.async_copy(msg_hbm.at[idx_b.at[s, 0]], rows_v.at[s],
                         sem_g[s])

    def gather_wait(s):
        pltpu.make_async_copy(msg_hbm.at[idx_b.at[s, 0]], rows_v.at[s],
                              sem_g[s]).wait()

    def scatter(s):
        pltpu.sync_copy(rows_v.at[s], agg_sp.at[idx_b.at[s, 1]], add=True)

    idx_start(gbase, 0)
    idx_start(gbase + 1, 1)
    plsc.subcore_barrier()
    idx_wait(0)
    gather_start(0)

    def step(c, s):
        nx = 1 - s
        idx_wait(nx)
        gather_start(nx)
        gather_wait(s)
        scatter(s)
        idx_start(gbase + jnp.minimum(c + 2, GPW - 1), s)

    def pair(cc, carry):
        step(2 * cc, 0)
        step(2 * cc + 1, 1)
        return carry

    lax.fori_loop(0, (GPW - 2) // 2, pair, 0)

    idx_wait(1)
    gather_start(1)
    gather_wait(0)
    scatter(0)
    gather_wait(1)
    scatter(1)

    @pl.when(sid < GEXTRA // NC)
    def _():
        idx_start(NW * GPW + cid * (GEXTRA // NC) + sid, 0)
        idx_wait(0)
        gather_start(0)
        gather_wait(0)
        scatter(0)

    plsc.subcore_barrier()

    pltpu.sync_copy(agg_sp.at[pl.ds(sid * RPT, RPT)],
                    out_hbm.at[cid, pl.ds(sid * RPT, RPT)])


_sc_scatter = functools.partial(
    pl.kernel,
    out_type=jax.ShapeDtypeStruct((NC, NPAD, H), jnp.float32),
    mesh=plsc.VectorSubcoreMesh(core_axis_name="c", subcore_axis_name="s"),
    scratch_types=[
        pltpu.VMEM((2, 2, K), jnp.int32),
        pltpu.VMEM((2, K, H), jnp.float32),
        pltpu.VMEM_SHARED((NPAD, H), jnp.float32),
        pltpu.SemaphoreType.DMA,
        pltpu.SemaphoreType.DMA,
        pltpu.SemaphoreType.DMA,
        pltpu.SemaphoreType.DMA,
    ],
)(_sc_body)


def kernel(x, edge_index, W1, b1, W2, b2, Wc, bc, W3, b3, W4, b4):
    b1r, b2r, bcr = b1.reshape(1, H), b2.reshape(1, H), bc.reshape(1, H)
    b3r, b4r = b3.reshape(1, H), b4.reshape(1, C_OUT)
    h, msg = _pre(x, W1, b1r, W2, b2r, Wc, bcr)
    agg2 = _sc_scatter(msg, edge_index)
    return _post(h, agg2, W3[:H], W3[H:], b3r, W4, b4r)
